# Initial kernel scaffold; baseline (speedup 1.0000x reference)
import jax
import jax.numpy as jnp
from jax.experimental import pallas as pl


def kernel(x, W1, b1, W2):
    raise NotImplementedError("write your pallas kernel here")


# speedup vs baseline: 1.4580x; 1.4580x over previous
import jax
import jax.numpy as jnp
from jax import lax
from jax.experimental import pallas as pl
from jax.experimental.pallas import tpu as pltpu

D_MODEL = 4096
D_HIDDEN = 2048
NUM_EXPERTS = 64
Z_LOSS_COEF = 0.001
BALANCE_LOSS_COEF = 0.01

BLOCK_T = 512


def _router_kernel(x_ref, w1_ref, b1_ref, w2_ref,
                   depth_ref, aux_ref,
                   probs_acc, counts_acc, lse2_acc):
    t = pl.program_id(0)
    nt = pl.num_programs(0)
    n_tokens = nt * BLOCK_T

    @pl.when(t == 0)
    def _init():
        probs_acc[...] = jnp.zeros_like(probs_acc)
        counts_acc[...] = jnp.zeros_like(counts_acc)
        lse2_acc[0, 0] = 0.0

    xb = x_ref[...]
    h = jnp.dot(xb, w1_ref[...], preferred_element_type=jnp.float32)
    h = h + b1_ref[...]
    h = h * jax.nn.sigmoid(h)
    logits = jnp.dot(h, w2_ref[...], preferred_element_type=jnp.float32)

    m = jnp.max(logits, axis=1, keepdims=True)
    e = jnp.exp(logits - m)
    s = jnp.sum(e, axis=1, keepdims=True)
    probs = e / s
    lse = m + jnp.log(s)

    pm = jnp.max(probs, axis=1, keepdims=True)
    ii = lax.broadcasted_iota(jnp.int32, probs.shape, 1)
    sel = jnp.min(jnp.where(probs == pm, ii, NUM_EXPERTS), axis=1,
                  keepdims=True)
    depth_ref[...] = jnp.reshape(sel[:, 0] + 1, depth_ref.shape)

    probs_acc[...] += jnp.sum(probs, axis=0, keepdims=True)
    onehot = (ii == sel).astype(jnp.float32)
    counts_acc[...] += jnp.sum(onehot, axis=0, keepdims=True)
    lse2_acc[0, 0] += jnp.sum(lse * lse)

    @pl.when(t == nt - 1)
    def _finalize():
        z_loss = lse2_acc[0, 0] / n_tokens
        bal = NUM_EXPERTS * jnp.sum(
            (counts_acc[...] / n_tokens) * (probs_acc[...] / n_tokens))
        aux_ref[0, 0] = Z_LOSS_COEF * z_loss + BALANCE_LOSS_COEF * bal


def kernel(x, W1, b1, W2):
    batch_size, seq_len, d_model = x.shape
    n_tokens = batch_size * seq_len
    nt = n_tokens // BLOCK_T
    x_flat = x.reshape(n_tokens, d_model)
    b1_2d = b1.reshape(1, D_HIDDEN)

    depths, aux = pl.pallas_call(
        _router_kernel,
        grid=(nt,),
        in_specs=[
            pl.BlockSpec((BLOCK_T, D_MODEL), lambda t: (t, 0)),
            pl.BlockSpec((D_MODEL, D_HIDDEN), lambda t: (0, 0)),
            pl.BlockSpec((1, D_HIDDEN), lambda t: (0, 0)),
            pl.BlockSpec((D_HIDDEN, NUM_EXPERTS), lambda t: (0, 0)),
        ],
        out_specs=[
            pl.BlockSpec((1, 1, BLOCK_T), lambda t: (t, 0, 0)),
            pl.BlockSpec(memory_space=pltpu.SMEM),
        ],
        out_shape=[
            jax.ShapeDtypeStruct((nt, 1, BLOCK_T), jnp.int32),
            jax.ShapeDtypeStruct((1, 1), jnp.float32),
        ],
        scratch_shapes=[
            pltpu.VMEM((1, NUM_EXPERTS), jnp.float32),
            pltpu.VMEM((1, NUM_EXPERTS), jnp.float32),
            pltpu.SMEM((1, 1), jnp.float32),
        ],
        compiler_params=pltpu.CompilerParams(
            dimension_semantics=("arbitrary",),
        ),
    )(x_flat, W1, b1_2d, W2)

    assigned_depths = depths.reshape(batch_size, seq_len)
    aux_loss = aux.reshape(())
    return assigned_depths, aux_loss


# speedup vs baseline: 1.5474x; 1.0613x over previous
import jax
import jax.numpy as jnp
from jax import lax
from jax.experimental import pallas as pl
from jax.experimental.pallas import tpu as pltpu

D_MODEL = 4096
D_HIDDEN = 2048
NUM_EXPERTS = 64
Z_LOSS_COEF = 0.001
BALANCE_LOSS_COEF = 0.01

BLOCK_T = 512


def _router_kernel(x_ref, w1_ref, b1_ref, w2_ref,
                   depth_ref, aux_ref,
                   logits_sc, probs_acc, counts_acc, lse2_acc):
    t = pl.program_id(0)
    nt = pl.num_programs(0)
    n_tokens = nt * BLOCK_T

    @pl.when(t == 0)
    def _init():
        probs_acc[...] = jnp.zeros_like(probs_acc)
        counts_acc[...] = jnp.zeros_like(counts_acc)
        lse2_acc[0, 0] = 0.0
        logits_sc[...] = jnp.zeros_like(logits_sc)

    prev_logits = logits_sc[...]

    def epilogue(logits, row, scale):
        m = jnp.max(logits, axis=1, keepdims=True)
        e = jnp.exp(logits - m)
        s = jnp.sum(e, axis=1, keepdims=True)
        probs = e / s
        lse = m + jnp.log(s)

        pm = jnp.max(probs, axis=1, keepdims=True)
        ii = lax.broadcasted_iota(jnp.int32, probs.shape, 1)
        sel = jnp.min(jnp.where(probs == pm, ii, NUM_EXPERTS), axis=1,
                      keepdims=True)
        depth_ref[pl.ds(row, 1), :, :] = jnp.reshape(sel[:, 0] + 1,
                                                     (1, 1, BLOCK_T))
        probs_acc[...] += scale * jnp.sum(probs, axis=0, keepdims=True)
        onehot = (ii == sel).astype(jnp.float32)
        counts_acc[...] += scale * jnp.sum(onehot, axis=0, keepdims=True)
        lse2_acc[0, 0] += scale * jnp.sum(lse * lse)

    epilogue(prev_logits, jnp.maximum(t - 1, 0),
             jnp.where(t > 0, 1.0, 0.0).astype(jnp.float32))

    xb = x_ref[...]
    h = jnp.dot(xb, w1_ref[...], preferred_element_type=jnp.float32)
    h = h + b1_ref[...]
    h = h * jax.nn.sigmoid(h)
    logits_sc[...] = jnp.dot(h, w2_ref[...],
                             preferred_element_type=jnp.float32)

    @pl.when(t == nt - 1)
    def _finalize():
        epilogue(logits_sc[...], t, jnp.float32(1.0))
        z_loss = lse2_acc[0, 0] / n_tokens
        bal = NUM_EXPERTS * jnp.sum(
            (counts_acc[...] / n_tokens) * (probs_acc[...] / n_tokens))
        aux_ref[0, 0] = Z_LOSS_COEF * z_loss + BALANCE_LOSS_COEF * bal


def kernel(x, W1, b1, W2):
    batch_size, seq_len, d_model = x.shape
    n_tokens = batch_size * seq_len
    nt = n_tokens // BLOCK_T
    x_flat = x.reshape(n_tokens, d_model)
    b1_2d = b1.reshape(1, D_HIDDEN)

    depths, aux = pl.pallas_call(
        _router_kernel,
        grid=(nt,),
        in_specs=[
            pl.BlockSpec((BLOCK_T, D_MODEL), lambda t: (t, 0)),
            pl.BlockSpec((D_MODEL, D_HIDDEN), lambda t: (0, 0)),
            pl.BlockSpec((1, D_HIDDEN), lambda t: (0, 0)),
            pl.BlockSpec((D_HIDDEN, NUM_EXPERTS), lambda t: (0, 0)),
        ],
        out_specs=[
            pl.BlockSpec((nt, 1, BLOCK_T), lambda t: (0, 0, 0)),
            pl.BlockSpec(memory_space=pltpu.SMEM),
        ],
        out_shape=[
            jax.ShapeDtypeStruct((nt, 1, BLOCK_T), jnp.int32),
            jax.ShapeDtypeStruct((1, 1), jnp.float32),
        ],
        scratch_shapes=[
            pltpu.VMEM((BLOCK_T, NUM_EXPERTS), jnp.float32),
            pltpu.VMEM((1, NUM_EXPERTS), jnp.float32),
            pltpu.VMEM((1, NUM_EXPERTS), jnp.float32),
            pltpu.SMEM((1, 1), jnp.float32),
        ],
        compiler_params=pltpu.CompilerParams(
            dimension_semantics=("arbitrary",),
        ),
    )(x_flat, W1, b1_2d, W2)

    assigned_depths = depths.reshape(batch_size, seq_len)
    aux_loss = aux.reshape(())
    return assigned_depths, aux_loss
